# Initial kernel scaffold; baseline (speedup 1.0000x reference)
import jax
import jax.numpy as jnp
from jax.experimental import pallas as pl


def kernel(adjacency_indices, adjacency_values, fea_input, weight, bias):
    raise NotImplementedError("write your pallas kernel here")


# speedup vs baseline: 4.4090x; 4.4090x over previous
import functools

import jax
import jax.numpy as jnp
from jax import lax
from jax.experimental import pallas as pl
from jax.experimental.pallas import tpu as pltpu
from jax.experimental.pallas import tpu_sc as plsc

NC = 2
NS = 16
L = 16
NW = NC * NS


def _matmul_tc(fea, weight):
    n, d_in = fea.shape
    d_out = weight.shape[1]
    bn = 1000
    assert n % bn == 0

    def body(x_ref, w_ref, o_ref):
        o_ref[...] = jnp.dot(x_ref[...], w_ref[...],
                             preferred_element_type=jnp.float32)

    return pl.pallas_call(
        body,
        grid=(n // bn,),
        in_specs=[pl.BlockSpec((bn, d_in), lambda i: (i, 0)),
                  pl.BlockSpec((d_in, d_out), lambda i: (0, 0))],
        out_specs=pl.BlockSpec((bn, d_out), lambda i: (i, 0)),
        out_shape=jax.ShapeDtypeStruct((n, d_out), jnp.float32),
    )(fea, weight)


def _combine_tc(partials, bias2d):
    _, n, d_out = partials.shape
    bn = 1000
    assert n % bn == 0

    def body(p_ref, b_ref, o_ref):
        o_ref[...] = p_ref[0] + p_ref[1] + b_ref[...]

    return pl.pallas_call(
        body,
        grid=(n // bn,),
        in_specs=[pl.BlockSpec((2, bn, d_out), lambda i: (0, i, 0)),
                  pl.BlockSpec((1, d_out), lambda i: (0, 0))],
        out_specs=pl.BlockSpec((bn, d_out), lambda i: (i, 0)),
        out_shape=jax.ShapeDtypeStruct((n, d_out), jnp.float32),
    )(partials, bias2d)


def _spmm_sc(rows, cols, vals, support, zeros_tile):
    n, d_out = support.shape
    e = rows.shape[0]
    k = 80
    epw = e // NW
    nch = epw // k
    zb = (n // NS) // 8 * 8
    tail = n - NS * zb
    assert epw * NW == e and nch * k == epw
    assert epw % 8 == 0 and k % 8 == 0 and d_out % L == 0
    assert tail % 8 == 0 and 0 < tail <= zb

    mesh = plsc.VectorSubcoreMesh(core_axis_name="c", subcore_axis_name="s")

    @functools.partial(
        pl.kernel,
        out_type=jax.ShapeDtypeStruct((NC, n, d_out), jnp.float32),
        mesh=mesh,
        scratch_types=[
            pltpu.VMEM((k,), jnp.int32),
            pltpu.VMEM((k,), jnp.int32),
            pltpu.VMEM((k,), jnp.float32),
            pltpu.VMEM((k, d_out), jnp.float32),
            pltpu.VMEM_SHARED((n, d_out), jnp.float32),
            pltpu.SemaphoreType.DMA,
        ],
    )
    def spmm(rows_hbm, cols_hbm, vals_hbm, support_hbm, zeros_hbm, out_hbm,
             rowv, colv, valv, gat, partial, sem):
        c = lax.axis_index("c")
        s = lax.axis_index("s")
        w = c * NS + s
        pltpu.sync_copy(zeros_hbm.at[pl.ds(0, zb)],
                        partial.at[pl.ds(s * zb, zb)])

        @pl.when(s == NS - 1)
        def _zero_tail():
            pltpu.sync_copy(zeros_hbm.at[pl.ds(0, tail)],
                            partial.at[pl.ds(NS * zb, tail)])

        plsc.subcore_barrier()
        ebase = w * epw

        def chunk_body(i, carry):
            base = ebase + i * k
            pltpu.sync_copy(cols_hbm.at[pl.ds(base, k)], colv)
            pltpu.sync_copy(rows_hbm.at[pl.ds(base, k)], rowv)
            pltpu.sync_copy(vals_hbm.at[pl.ds(base, k)], valv)
            pltpu.async_copy(support_hbm.at[colv], gat, sem).wait()

            def scale_group(g, carry2):
                base16 = g * L
                vvec = valv[pl.ds(base16, L)]
                dnums = lax.GatherDimensionNumbers(
                    offset_dims=(), collapsed_slice_dims=(0,),
                    start_index_map=(0,))
                for ei in range(L):
                    vs = lax.gather(
                        vvec, jnp.full((L, 1), ei, jnp.int32), dnums,
                        slice_sizes=(1,),
                        mode=lax.GatherScatterMode.PROMISE_IN_BOUNDS)
                    row = base16 + ei
                    for j in range(d_out // L):
                        gat[row, pl.ds(j * L, L)] = (
                            gat[row, pl.ds(j * L, L)] * vs)
                return carry2

            lax.fori_loop(0, k // L, scale_group, 0)
            pltpu.sync_copy(gat, partial.at[rowv], add=True)
            return carry

        lax.fori_loop(0, nch, chunk_body, 0)
        plsc.subcore_barrier()
        pltpu.sync_copy(partial.at[pl.ds(s * zb, zb)],
                        out_hbm.at[c, pl.ds(s * zb, zb)])

        @pl.when(s == NS - 1)
        def _copy_tail():
            pltpu.sync_copy(partial.at[pl.ds(NS * zb, tail)],
                            out_hbm.at[c, pl.ds(NS * zb, tail)])

    return spmm(rows, cols, vals, support, zeros_tile)


def kernel(adjacency_indices, adjacency_values, fea_input, weight, bias):
    rows = adjacency_indices[0].astype(jnp.int32)
    cols = adjacency_indices[1].astype(jnp.int32)
    support = _matmul_tc(fea_input, weight)
    zeros_tile = jnp.zeros((fea_input.shape[0] // NS // 8 * 8,
                            weight.shape[1]), jnp.float32)
    partials = _spmm_sc(rows, cols, adjacency_values, support, zeros_tile)
    return _combine_tc(partials, bias.reshape(1, -1))


# speedup vs baseline: 7.8196x; 1.7735x over previous
import functools

import jax
import jax.numpy as jnp
from jax import lax
from jax.experimental import pallas as pl
from jax.experimental.pallas import tpu as pltpu
from jax.experimental.pallas import tpu_sc as plsc

NC = 2
NS = 16
L = 16
K = 112


def _matmul_tc(fea, weight):
    n, d_in = fea.shape
    d_out = weight.shape[1]
    dh = d_out // NC
    bn = 1000
    assert n % bn == 0 and d_out % NC == 0

    def body(x_ref, w_ref, o_ref):
        r = jnp.dot(x_ref[...], w_ref[...], preferred_element_type=jnp.float32)
        for c in range(NC):
            o_ref[c, :, :] = r[:, c * dh:(c + 1) * dh]

    return pl.pallas_call(
        body,
        grid=(n // bn,),
        in_specs=[pl.BlockSpec((bn, d_in), lambda i: (i, 0)),
                  pl.BlockSpec((d_in, d_out), lambda i: (0, 0))],
        out_specs=pl.BlockSpec((NC, bn, dh), lambda i: (0, i, 0)),
        out_shape=jax.ShapeDtypeStruct((NC, n, dh), jnp.float32),
    )(fea, weight)


def _combine_tc(partials, bias):
    _, n, dh = partials.shape
    d_out = NC * dh
    bn = 1000
    assert n % bn == 0

    def body(p_ref, b_ref, o_ref):
        for c in range(NC):
            o_ref[:, c * dh:(c + 1) * dh] = p_ref[c] + b_ref[c]

    return pl.pallas_call(
        body,
        grid=(n // bn,),
        in_specs=[pl.BlockSpec((NC, bn, dh), lambda i: (0, i, 0)),
                  pl.BlockSpec((NC, 1, dh), lambda i: (0, 0, 0))],
        out_specs=pl.BlockSpec((bn, d_out), lambda i: (i, 0)),
        out_shape=jax.ShapeDtypeStruct((n, d_out), jnp.float32),
    )(partials, bias.reshape(NC, 1, dh))


def _spmm_sc(rows3, cols3, vals3, support2, zeros_tile):
    _, n, dh = support2.shape
    ns, nch, k = rows3.shape
    zb = (n // NS) // 8 * 8
    tail = n - NS * zb
    assert ns == NS and k == K and nch % 2 == 0 and k % L == 0
    assert dh % L == 0 and tail % 8 == 0 and 0 < tail <= zb

    mesh = plsc.VectorSubcoreMesh(core_axis_name="c", subcore_axis_name="s")

    @functools.partial(
        pl.kernel,
        out_type=jax.ShapeDtypeStruct((NC, n, dh), jnp.float32),
        mesh=mesh,
        scratch_types=[
            pltpu.VMEM((nch, k), jnp.int32),
            pltpu.VMEM((nch, k), jnp.int32),
            pltpu.VMEM((nch, k), jnp.float32),
            pltpu.VMEM((k, dh), jnp.float32),
            pltpu.VMEM((k, dh), jnp.float32),
            pltpu.VMEM((k, dh), jnp.float32),
            pltpu.VMEM((k, dh), jnp.float32),
            pltpu.VMEM_SHARED((n, dh), jnp.float32),
            pltpu.SemaphoreType.DMA,
            pltpu.SemaphoreType.DMA,
            pltpu.SemaphoreType.DMA,
            pltpu.SemaphoreType.DMA,
        ],
        compiler_params=pltpu.CompilerParams(use_tc_tiling_on_sc=False),
    )
    def spmm(rows_hbm, cols_hbm, vals_hbm, support_hbm, zeros_hbm, out_hbm,
             rowv, colv, valv, gat0, gat1, sct0, sct1, partial,
             gsem0, gsem1, ssem0, ssem1):
        c = lax.axis_index("c")
        s = lax.axis_index("s")
        gats, scts = (gat0, gat1), (sct0, sct1)
        gsems, ssems = (gsem0, gsem1), (ssem0, ssem1)

        pltpu.sync_copy(rows_hbm.at[s], rowv)
        pltpu.sync_copy(cols_hbm.at[s], colv)
        pltpu.sync_copy(vals_hbm.at[s], valv)
        pltpu.sync_copy(zeros_hbm.at[pl.ds(0, zb)],
                        partial.at[pl.ds(s * zb, zb)])

        @pl.when(s == NS - 1)
        def _zero_tail():
            pltpu.sync_copy(zeros_hbm.at[pl.ds(0, tail)],
                            partial.at[pl.ds(NS * zb, tail)])

        plsc.subcore_barrier()

        dnums = lax.GatherDimensionNumbers(
            offset_dims=(), collapsed_slice_dims=(0,), start_index_map=(0,))

        def scale(i, src, dst):
            def group(g, carry):
                base16 = g * L
                vvec = valv[i, pl.ds(base16, L)]
                for ei in range(L):
                    vs = lax.gather(
                        vvec, jnp.full((L, 1), ei, jnp.int32), dnums,
                        slice_sizes=(1,),
                        mode=lax.GatherScatterMode.PROMISE_IN_BOUNDS)
                    row = base16 + ei
                    for j in range(dh // L):
                        dst[row, pl.ds(j * L, L)] = (
                            src[row, pl.ds(j * L, L)] * vs)
                return carry

            plsc.parallel_loop(0, k // L, 1, unroll=2,
                               carry=jnp.int32(0))(group)

        for b in range(2):
            pltpu.async_copy(support_hbm.at[c].at[colv.at[b]],
                             gats[b], gsems[b])

        def body(p, carry):
            for b in range(2):
                i = 2 * p + b
                pltpu.make_async_copy(
                    support_hbm.at[c].at[colv.at[i]], gats[b],
                    gsems[b]).wait()

                @pl.when(i >= 2)
                def _wait_prev_scatter():
                    pltpu.make_async_copy(
                        scts[b], partial.at[rowv.at[i - 2]], ssems[b]).wait()

                scale(i, gats[b], scts[b])

                @pl.when(i + 2 < nch)
                def _start_next_gather():
                    pltpu.async_copy(
                        support_hbm.at[c].at[colv.at[i + 2]],
                        gats[b], gsems[b])

                pltpu.async_copy(
                    scts[b], partial.at[rowv.at[i]], ssems[b], add=True)
            return carry

        lax.fori_loop(0, nch // 2, body, 0)
        for b in range(2):
            pltpu.make_async_copy(
                scts[b], partial.at[rowv.at[nch - 2 + b]], ssems[b]).wait()

        plsc.subcore_barrier()
        pltpu.sync_copy(partial.at[pl.ds(s * zb, zb)],
                        out_hbm.at[c, pl.ds(s * zb, zb)])

        @pl.when(s == NS - 1)
        def _copy_tail():
            pltpu.sync_copy(partial.at[pl.ds(NS * zb, tail)],
                            out_hbm.at[c, pl.ds(NS * zb, tail)])

    return spmm(rows3, cols3, vals3, support2, zeros_tile)


def kernel(adjacency_indices, adjacency_values, fea_input, weight, bias):
    n = fea_input.shape[0]
    e = adjacency_values.shape[0]
    rows = adjacency_indices[0].astype(jnp.int32)
    cols = adjacency_indices[1].astype(jnp.int32)
    vals = adjacency_values.astype(jnp.float32)

    nch = -(-e // (NS * K))
    nch += nch % 2
    e_pad = NS * nch * K
    pad = e_pad - e
    if pad:
        rows = jnp.concatenate([rows, jnp.zeros((pad,), jnp.int32)])
        cols = jnp.concatenate([cols, jnp.zeros((pad,), jnp.int32)])
        vals = jnp.concatenate([vals, jnp.zeros((pad,), jnp.float32)])
    rows3 = rows.reshape(NS, nch, K)
    cols3 = cols.reshape(NS, nch, K)
    vals3 = vals.reshape(NS, nch, K)

    support2 = _matmul_tc(fea_input, weight)
    zeros_tile = jnp.zeros((n // NS // 8 * 8, weight.shape[1] // NC),
                           jnp.float32)
    partials = _spmm_sc(rows3, cols3, vals3, support2, zeros_tile)
    return _combine_tc(partials, bias)


# speedup vs baseline: 9.4556x; 1.2092x over previous
import functools

import jax
import jax.numpy as jnp
from jax import lax
from jax.experimental import pallas as pl
from jax.experimental.pallas import tpu as pltpu
from jax.experimental.pallas import tpu_sc as plsc

NC = 2
NS = 16
L = 16
K = 80


def _matmul_tc(fea, weight):
    n, d_in = fea.shape
    d_out = weight.shape[1]
    dh = d_out // NC
    bn = 1000
    assert n % bn == 0 and d_out % NC == 0

    def body(x_ref, w_ref, o_ref):
        r = jnp.dot(x_ref[...], w_ref[...], preferred_element_type=jnp.float32)
        for c in range(NC):
            o_ref[c, :, :] = r[:, c * dh:(c + 1) * dh]

    return pl.pallas_call(
        body,
        grid=(n // bn,),
        in_specs=[pl.BlockSpec((bn, d_in), lambda i: (i, 0)),
                  pl.BlockSpec((d_in, d_out), lambda i: (0, 0))],
        out_specs=pl.BlockSpec((NC, bn, dh), lambda i: (0, i, 0)),
        out_shape=jax.ShapeDtypeStruct((NC, n, dh), jnp.float32),
    )(fea, weight)


def _combine_tc(partials, bias):
    _, n, dh = partials.shape
    d_out = NC * dh
    bn = 1000
    assert n % bn == 0

    def body(p_ref, b_ref, o_ref):
        for c in range(NC):
            o_ref[:, c * dh:(c + 1) * dh] = p_ref[c] + b_ref[c]

    return pl.pallas_call(
        body,
        grid=(n // bn,),
        in_specs=[pl.BlockSpec((NC, bn, dh), lambda i: (0, i, 0)),
                  pl.BlockSpec((NC, 1, dh), lambda i: (0, 0, 0))],
        out_specs=pl.BlockSpec((bn, d_out), lambda i: (i, 0)),
        out_shape=jax.ShapeDtypeStruct((n, d_out), jnp.float32),
    )(partials, bias.reshape(NC, 1, dh))


def _spmm_sc(rows3, cols3, vals3, support2, zeros_tile):
    _, n, dh = support2.shape
    ns, nch, k = rows3.shape
    zb = (n // NS) // 8 * 8
    tail = n - NS * zb
    fullg = k // L
    assert ns == NS and k == K and nch % 2 == 0 and k % L == 0
    assert dh % L == 0 and tail % 8 == 0 and 0 < tail <= zb

    mesh = plsc.VectorSubcoreMesh(core_axis_name="c", subcore_axis_name="s")

    @functools.partial(
        pl.kernel,
        out_type=jax.ShapeDtypeStruct((NC, n, dh), jnp.float32),
        mesh=mesh,
        scratch_types=[
            pltpu.VMEM((nch, k), jnp.int32),
            pltpu.VMEM((nch, k), jnp.int32),
            pltpu.VMEM((nch, k), jnp.float32),
            pltpu.VMEM((k, dh), jnp.float32),
            pltpu.VMEM((k, dh), jnp.float32),
            pltpu.VMEM((k, dh), jnp.float32),
            pltpu.VMEM((k, dh), jnp.float32),
            pltpu.VMEM_SHARED((n, dh), jnp.float32),
            pltpu.SemaphoreType.DMA,
            pltpu.SemaphoreType.DMA,
            pltpu.SemaphoreType.DMA,
            pltpu.SemaphoreType.DMA,
        ],
        compiler_params=pltpu.CompilerParams(use_tc_tiling_on_sc=False),
    )
    def spmm(rows_hbm, cols_hbm, vals_hbm, support_hbm, zeros_hbm, out_hbm,
             rowv, colv, valv, gat0, gat1, sct0, sct1, partial,
             gsem0, gsem1, ssem0, ssem1):
        c = lax.axis_index("c")
        s = lax.axis_index("s")
        gats, scts = (gat0, gat1), (sct0, sct1)
        gsems, ssems = (gsem0, gsem1), (ssem0, ssem1)

        cp_c = pltpu.async_copy(cols_hbm.at[s], colv, gsem0)
        cp_v = pltpu.async_copy(vals_hbm.at[s], valv, gsem1)
        cp_r = pltpu.async_copy(rows_hbm.at[s], rowv, ssem0)
        cp_z = pltpu.async_copy(zeros_hbm.at[pl.ds(0, zb)],
                                partial.at[pl.ds(s * zb, zb)], ssem1)
        cp_c.wait()

        dnums = lax.GatherDimensionNumbers(
            offset_dims=(), collapsed_slice_dims=(0,), start_index_map=(0,))

        def scale(i, src, dst):
            def group(g, carry):
                base16 = g * L
                vvec = valv[i, pl.ds(base16, L)]
                for ei in range(L):
                    vs = lax.gather(
                        vvec, jnp.full((L, 1), ei, jnp.int32), dnums,
                        slice_sizes=(1,),
                        mode=lax.GatherScatterMode.PROMISE_IN_BOUNDS)
                    row = base16 + ei
                    for j in range(dh // L):
                        dst[row, pl.ds(j * L, L)] = (
                            src[row, pl.ds(j * L, L)] * vs)
                return carry

            plsc.parallel_loop(0, fullg, 1, unroll=2,
                               carry=jnp.int32(0))(group)

        for b in range(2):
            pltpu.async_copy(support_hbm.at[c].at[colv.at[b]],
                             gats[b], gsems[b])
        cp_v.wait()
        cp_r.wait()
        cp_z.wait()

        @pl.when(s == NS - 1)
        def _zero_tail():
            pltpu.sync_copy(zeros_hbm.at[pl.ds(0, tail)],
                            partial.at[pl.ds(NS * zb, tail)])

        plsc.subcore_barrier()

        def body(p, carry):
            for b in range(2):
                i = 2 * p + b
                pltpu.make_async_copy(
                    support_hbm.at[c].at[colv.at[i]], gats[b],
                    gsems[b]).wait()

                @pl.when(i >= 2)
                def _wait_prev_scatter():
                    pltpu.make_async_copy(
                        scts[b], partial.at[rowv.at[i - 2]], ssems[b]).wait()

                scale(i, gats[b], scts[b])

                @pl.when(i + 2 < nch)
                def _start_next_gather():
                    pltpu.async_copy(
                        support_hbm.at[c].at[colv.at[i + 2]],
                        gats[b], gsems[b])

                pltpu.async_copy(
                    scts[b], partial.at[rowv.at[i]], ssems[b], add=True)
            return carry

        lax.fori_loop(0, nch // 2, body, 0)
        for b in range(2):
            pltpu.make_async_copy(
                scts[b], partial.at[rowv.at[nch - 2 + b]], ssems[b]).wait()

        plsc.subcore_barrier()
        pltpu.sync_copy(partial.at[pl.ds(s * zb, zb)],
                        out_hbm.at[c, pl.ds(s * zb, zb)])

        @pl.when(s == NS - 1)
        def _copy_tail():
            pltpu.sync_copy(partial.at[pl.ds(NS * zb, tail)],
                            out_hbm.at[c, pl.ds(NS * zb, tail)])

    return spmm(rows3, cols3, vals3, support2, zeros_tile)


def kernel(adjacency_indices, adjacency_values, fea_input, weight, bias):
    n = fea_input.shape[0]
    e = adjacency_values.shape[0]
    rows = adjacency_indices[0].astype(jnp.int32)
    cols = adjacency_indices[1].astype(jnp.int32)
    vals = adjacency_values.astype(jnp.float32)

    nch = -(-e // (NS * K))
    nch += nch % 2
    e_pad = NS * nch * K
    pad = e_pad - e
    if pad:
        rows = jnp.concatenate([rows, jnp.zeros((pad,), jnp.int32)])
        cols = jnp.concatenate([cols, jnp.zeros((pad,), jnp.int32)])
        vals = jnp.concatenate([vals, jnp.zeros((pad,), jnp.float32)])
    rows3 = rows.reshape(NS, nch, K)
    cols3 = cols.reshape(NS, nch, K)
    vals3 = vals.reshape(NS, nch, K)

    support2 = _matmul_tc(fea_input, weight)
    zeros_tile = jnp.zeros((n // NS // 8 * 8, weight.shape[1] // NC),
                           jnp.float32)
    partials = _spmm_sc(rows3, cols3, vals3, support2, zeros_tile)
    return _combine_tc(partials, bias)


# speedup vs baseline: 9.4679x; 1.0013x over previous
import functools

import jax
import jax.numpy as jnp
from jax import lax
from jax.experimental import pallas as pl
from jax.experimental.pallas import tpu as pltpu
from jax.experimental.pallas import tpu_sc as plsc

NC = 2
NS = 16
L = 16
K = 80


def _matmul_tc(fea, weight):
    n, d_in = fea.shape
    d_out = weight.shape[1]
    dh = d_out // NC
    bn = 1000
    assert n % bn == 0 and d_out % NC == 0

    def body(x_ref, w_ref, o_ref):
        r = jnp.dot(x_ref[...], w_ref[...], preferred_element_type=jnp.float32)
        for c in range(NC):
            o_ref[c, :, :] = r[:, c * dh:(c + 1) * dh]

    return pl.pallas_call(
        body,
        grid=(n // bn,),
        in_specs=[pl.BlockSpec((bn, d_in), lambda i: (i, 0)),
                  pl.BlockSpec((d_in, d_out), lambda i: (0, 0))],
        out_specs=pl.BlockSpec((NC, bn, dh), lambda i: (0, i, 0)),
        out_shape=jax.ShapeDtypeStruct((NC, n, dh), jnp.float32),
    )(fea, weight)


def _spmm_sc(rows3, cols3, vals3, support2, init_tile):
    _, n, dh = support2.shape
    ns, nch, k = rows3.shape
    zb = (n // NS) // 8 * 8
    tail = n - NS * zb
    fullg = k // L
    assert ns == NS and k == K and nch % 2 == 0 and k % L == 0
    assert dh % L == 0 and tail % 8 == 0 and 0 < tail <= zb

    mesh = plsc.VectorSubcoreMesh(core_axis_name="c", subcore_axis_name="s")

    @functools.partial(
        pl.kernel,
        out_type=jax.ShapeDtypeStruct((NC, n, dh), jnp.float32),
        mesh=mesh,
        scratch_types=[
            pltpu.VMEM((nch, k), jnp.int32),
            pltpu.VMEM((nch, k), jnp.int32),
            pltpu.VMEM((nch, k), jnp.float32),
            pltpu.VMEM((k, dh), jnp.float32),
            pltpu.VMEM((k, dh), jnp.float32),
            pltpu.VMEM((k, dh), jnp.float32),
            pltpu.VMEM((k, dh), jnp.float32),
            pltpu.VMEM_SHARED((n, dh), jnp.float32),
            pltpu.SemaphoreType.DMA,
            pltpu.SemaphoreType.DMA,
            pltpu.SemaphoreType.DMA,
            pltpu.SemaphoreType.DMA,
        ],
        compiler_params=pltpu.CompilerParams(use_tc_tiling_on_sc=False),
    )
    def spmm(rows_hbm, cols_hbm, vals_hbm, support_hbm, init_hbm, out_hbm,
             rowv, colv, valv, gat0, gat1, sct0, sct1, partial,
             gsem0, gsem1, ssem0, ssem1):
        c = lax.axis_index("c")
        s = lax.axis_index("s")
        gats, scts = (gat0, gat1), (sct0, sct1)
        gsems, ssems = (gsem0, gsem1), (ssem0, ssem1)

        cp_c = pltpu.async_copy(cols_hbm.at[s], colv, gsem0)
        cp_v = pltpu.async_copy(vals_hbm.at[s], valv, gsem1)
        cp_r = pltpu.async_copy(rows_hbm.at[s], rowv, ssem0)
        cp_z = pltpu.async_copy(init_hbm.at[c, pl.ds(0, zb)],
                                partial.at[pl.ds(s * zb, zb)], ssem1)
        cp_c.wait()

        dnums = lax.GatherDimensionNumbers(
            offset_dims=(), collapsed_slice_dims=(0,), start_index_map=(0,))

        def scale(i, src, dst):
            def group(g, carry):
                base16 = g * L
                vvec = valv[i, pl.ds(base16, L)]
                for ei in range(L):
                    vs = lax.gather(
                        vvec, jnp.full((L, 1), ei, jnp.int32), dnums,
                        slice_sizes=(1,),
                        mode=lax.GatherScatterMode.PROMISE_IN_BOUNDS)
                    row = base16 + ei
                    for j in range(dh // L):
                        dst[row, pl.ds(j * L, L)] = (
                            src[row, pl.ds(j * L, L)] * vs)
                return carry

            plsc.parallel_loop(0, fullg, 1, unroll=2,
                               carry=jnp.int32(0))(group)

        for b in range(2):
            pltpu.async_copy(support_hbm.at[c].at[colv.at[b]],
                             gats[b], gsems[b])
        cp_v.wait()
        cp_r.wait()
        cp_z.wait()

        @pl.when(s == NS - 1)
        def _init_tail():
            pltpu.sync_copy(init_hbm.at[c, pl.ds(0, tail)],
                            partial.at[pl.ds(NS * zb, tail)])

        plsc.subcore_barrier()

        def body(p, carry):
            for b in range(2):
                i = 2 * p + b
                pltpu.make_async_copy(
                    support_hbm.at[c].at[colv.at[i]], gats[b],
                    gsems[b]).wait()

                @pl.when(i >= 2)
                def _wait_prev_scatter():
                    pltpu.make_async_copy(
                        scts[b], partial.at[rowv.at[i - 2]], ssems[b]).wait()

                scale(i, gats[b], scts[b])

                @pl.when(i + 2 < nch)
                def _start_next_gather():
                    pltpu.async_copy(
                        support_hbm.at[c].at[colv.at[i + 2]],
                        gats[b], gsems[b])

                pltpu.async_copy(
                    scts[b], partial.at[rowv.at[i]], ssems[b], add=True)
            return carry

        lax.fori_loop(0, nch // 2, body, 0)
        for b in range(2):
            pltpu.make_async_copy(
                scts[b], partial.at[rowv.at[nch - 2 + b]], ssems[b]).wait()

        plsc.subcore_barrier()
        pltpu.sync_copy(partial.at[pl.ds(s * zb, zb)],
                        out_hbm.at[c, pl.ds(s * zb, zb)])

        @pl.when(s == NS - 1)
        def _copy_tail():
            pltpu.sync_copy(partial.at[pl.ds(NS * zb, tail)],
                            out_hbm.at[c, pl.ds(NS * zb, tail)])

    return spmm(rows3, cols3, vals3, support2, init_tile)


def kernel(adjacency_indices, adjacency_values, fea_input, weight, bias):
    n = fea_input.shape[0]
    e = adjacency_values.shape[0]
    rows = adjacency_indices[0].astype(jnp.int32)
    cols = adjacency_indices[1].astype(jnp.int32)
    vals = adjacency_values.astype(jnp.float32)

    nch = -(-e // (NS * K))
    nch += nch % 2
    e_pad = NS * nch * K
    pad = e_pad - e
    if pad:
        rows = jnp.concatenate([rows, jnp.zeros((pad,), jnp.int32)])
        cols = jnp.concatenate([cols, jnp.zeros((pad,), jnp.int32)])
        vals = jnp.concatenate([vals, jnp.zeros((pad,), jnp.float32)])
    rows3 = rows.reshape(NS, nch, K)
    cols3 = cols.reshape(NS, nch, K)
    vals3 = vals.reshape(NS, nch, K)

    support2 = _matmul_tc(fea_input, weight)
    d_out = weight.shape[1]
    zb = n // NS // 8 * 8
    init_tile = jnp.broadcast_to(
        bias.astype(jnp.float32).reshape(NC, 1, d_out // NC),
        (NC, zb, d_out // NC))
    partials = _spmm_sc(rows3, cols3, vals3, support2, init_tile)
    return jnp.swapaxes(partials, 0, 1).reshape(n, d_out)
